# baseline (speedup 1.0000x reference)
import functools

import jax
import jax.numpy as jnp
from jax import lax
from jax.experimental import pallas as pl
from jax.experimental.pallas import tpu as pltpu
from jax.experimental.pallas import tpu_sc as plsc

N = 50000
E = 800000
EMB = 16
HID = 100
NUM_GRAPHS = 512

NP = 50176
HP = 128
NPG = 640
VP = 50176

NCORES = 2
NSUB = 16
NTILES = NCORES * NSUB
ROWS_PER_TILE = NP // NTILES
GCHUNK = 112
NGCH = ROWS_PER_TILE // GCHUNK

ZROWS = 112
RANGES = 8
RSIZE = NP // RANGES
PASSES = RANGES // NCORES
ZPS = RSIZE // NSUB
ZCH = 56
NZCH = ZPS // ZCH
EPS = E // NSUB
ECH = 256
NSLOT = 2
BCH = NSLOT * ECH
NBIG = EPS // BCH
TAILE = EPS - NBIG * BCH
IGN = -1


@functools.cache
def _mesh():
    return plsc.VectorSubcoreMesh(core_axis_name="c", subcore_axis_name="s",
                                  num_cores=NCORES, num_subcores=NSUB)


def _emb_body(i0, i1, i2, i3, i4, stacked, out_h, idx_v, rowbuf, sem):
    c = lax.axis_index("c")
    s = lax.axis_index("s")
    base = (s * NCORES + c) * ROWS_PER_TILE
    idxs = (i0, i1, i2, i3, i4)

    def body(j, _):
        r0 = base + j * GCHUNK
        pltpu.sync_copy(idxs[0].at[pl.ds(r0, GCHUNK)], idx_v)
        pltpu.async_copy(stacked.at[idx_v], rowbuf, sem).wait()
        for t in range(1, 5):
            pltpu.sync_copy(idxs[t].at[pl.ds(r0, GCHUNK)], idx_v)
            pltpu.async_copy(stacked.at[idx_v], rowbuf, sem, add=True).wait()
        pltpu.sync_copy(rowbuf, out_h.at[pl.ds(r0, GCHUNK)])
        return 0
    lax.fori_loop(0, NGCH, body, 0)


@functools.cache
def _emb_gather():
    return pl.kernel(
        _emb_body,
        out_type=jax.ShapeDtypeStruct((NP, HP), jnp.float32),
        mesh=_mesh(),
        scratch_types=[
            pltpu.VMEM((GCHUNK,), jnp.int32),
            pltpu.VMEM((GCHUNK, HP), jnp.float32),
            pltpu.SemaphoreType.DMA,
        ],
    )


def _agg_body(h_h, src_h, dst_h, zeros_h, msum_h,
              sbufL, dbufL, si0, si1, si2, si3, di0, di1, di2, di3, rows,
              g0, g1, acc):
    gs = (g0, g1)
    sis = ((si0, si1), (si2, si3))
    dis = ((di0, di1), (di2, di3))
    c = lax.axis_index("c")
    s = lax.axis_index("s")
    eb = s * EPS

    def load_big(j):
        e0 = eb + j * BCH
        pltpu.sync_copy(src_h.at[pl.ds(e0, BCH)], sbufL)
        pltpu.sync_copy(dst_h.at[pl.ds(e0, BCH)], dbufL)

    def mask_slot(lo, q, sl, nv):
        rs = sis[q][sl]
        rd = dis[q][sl]
        for i in range(nv):
            off = sl * ECH + i * 16
            d = dbufL[pl.ds(off, 16)]
            sv = sbufL[pl.ds(off, 16)]
            m = (d >= lo) & (d < lo + RSIZE)
            rs[pl.ds(i * 16, 16)] = jnp.where(m, sv, IGN)
            rd[pl.ds(i * 16, 16)] = jnp.where(m, d - lo, IGN)
        for i in range(nv, ECH // 16):
            rs[pl.ds(i * 16, 16)] = jnp.full((16,), IGN, jnp.int32)
            rd[pl.ds(i * 16, 16)] = jnp.full((16,), IGN, jnp.int32)

    def gsrc(q, sl):
        return h_h.at[plsc.Indices(sis[q][sl], ignored_value=IGN)]

    def issue_gather(q, sl):
        pltpu.async_copy(gsrc(q, sl), rows.at[sl], gs[sl])

    def wait_gather(q, sl):
        pltpu.make_async_copy(gsrc(q, sl), rows.at[sl], gs[sl]).wait()

    def do_add(q, sl):
        pltpu.sync_copy(rows.at[sl],
                        acc.at[plsc.Indices(dis[q][sl], ignored_value=IGN)],
                        add=True)

    for p in range(PASSES):
        lo = (2 * p + c) * RSIZE

        def zbody(j, _):
            pltpu.sync_copy(zeros_h.at[pl.ds(0, ZCH)],
                            acc.at[pl.ds(s * ZPS + j * ZCH, ZCH)])
            return 0
        lax.fori_loop(0, NZCH, zbody, 0)
        plsc.subcore_barrier()

        load_big(0)
        for sl in range(NSLOT):
            mask_slot(lo, 0, sl, ECH // 16)
            issue_gather(0, sl)

        def big_iter(j, q):
            load_big(j)
            for sl in range(NSLOT):
                mask_slot(lo, q, sl, ECH // 16)
            for sl in range(NSLOT):
                wait_gather(1 - q, sl)
                do_add(1 - q, sl)
                issue_gather(q, sl)

        pairs = (NBIG - 1) // 2
        def mbody(k, _):
            j = 1 + 2 * k
            big_iter(j, 1)
            big_iter(j + 1, 0)
            return 0
        lax.fori_loop(0, pairs, mbody, 0)
        if (NBIG - 1) - 2 * pairs:
            big_iter(NBIG - 1, 1)
            last_q = 1
        else:
            last_q = 0

        for sl in range(NSLOT):
            wait_gather(last_q, sl)
            do_add(last_q, sl)

        e0 = eb + NBIG * BCH
        pltpu.sync_copy(src_h.at[pl.ds(e0, TAILE)], sbufL.at[pl.ds(0, TAILE)])
        pltpu.sync_copy(dst_h.at[pl.ds(e0, TAILE)], dbufL.at[pl.ds(0, TAILE)])
        tv = (ECH // 16, (TAILE - ECH) // 16)
        for sl in range(2):
            mask_slot(lo, 0, sl, tv[sl])
            issue_gather(0, sl)
        for sl in range(2):
            wait_gather(0, sl)
            do_add(0, sl)
        plsc.subcore_barrier()

        pltpu.sync_copy(acc.at[pl.ds(s * ZPS, ZPS)],
                        msum_h.at[pl.ds(lo + s * ZPS, ZPS)])
        plsc.subcore_barrier()


@functools.cache
def _edge_agg():
    return pl.kernel(
        _agg_body,
        out_type=jax.ShapeDtypeStruct((NP, HP), jnp.float32),
        mesh=_mesh(),
        scratch_types=(
            [pltpu.VMEM((BCH,), jnp.int32)] * 2
            + [pltpu.VMEM((ECH,), jnp.int32)] * 8
            + [
                pltpu.VMEM((NSLOT, ECH, HP), jnp.float32),
                pltpu.SemaphoreType.DMA,
                pltpu.SemaphoreType.DMA,
                pltpu.VMEM_SHARED((RSIZE, HP), jnp.float32),
            ]
        ),
    )


def _pool_body(hn_h, batch_h, zeros_h, gpart_h, bidx, rows_v, sem, acc):
    c = lax.axis_index("c")
    s = lax.axis_index("s")
    base = (s * NCORES + c) * ROWS_PER_TILE
    gz = NPG // NSUB
    pltpu.sync_copy(zeros_h.at[pl.ds(0, gz)], acc.at[pl.ds(s * gz, gz)])
    plsc.subcore_barrier()

    def body(j, _):
        r0 = base + j * GCHUNK
        pltpu.sync_copy(batch_h.at[pl.ds(r0, GCHUNK)], bidx)
        pltpu.sync_copy(hn_h.at[pl.ds(r0, GCHUNK)], rows_v)
        pltpu.sync_copy(rows_v, acc.at[bidx], add=True)
        return 0
    lax.fori_loop(0, NGCH, body, 0)
    plsc.subcore_barrier()
    pltpu.sync_copy(acc.at[pl.ds(s * gz, gz)],
                    gpart_h.at[c, pl.ds(s * gz, gz)])


@functools.cache
def _pool():
    return pl.kernel(
        _pool_body,
        out_type=jax.ShapeDtypeStruct((NCORES, NPG, HP), jnp.float32),
        mesh=_mesh(),
        scratch_types=[
            pltpu.VMEM((GCHUNK,), jnp.int32),
            pltpu.VMEM((GCHUNK, HP), jnp.float32),
            pltpu.SemaphoreType.DMA,
            pltpu.VMEM_SHARED((NPG, HP), jnp.float32),
        ],
    )


def _mlin_body(g_ref, p_ref, wt_ref, w0_ref, b_ref, o_ref):
    acc = jnp.dot(g_ref[...], wt_ref[...], preferred_element_type=jnp.float32)
    o_ref[...] = acc + p_ref[...] * w0_ref[...] + b_ref[...]


def _msg_linear(gath, price, wt, w0, b):
    blk = 256
    return pl.pallas_call(
        _mlin_body,
        grid=(NP // blk,),
        in_specs=[
            pl.BlockSpec((blk, HP), lambda i: (i, 0)),
            pl.BlockSpec((blk, 1), lambda i: (i, 0)),
            pl.BlockSpec((HP, HP), lambda i: (0, 0)),
            pl.BlockSpec((1, HP), lambda i: (0, 0)),
            pl.BlockSpec((1, HP), lambda i: (0, 0)),
        ],
        out_specs=pl.BlockSpec((blk, HP), lambda i: (i, 0)),
        out_shape=jax.ShapeDtypeStruct((NP, HP), jnp.float32),
    )(gath, price, wt, w0, b)


def _gru_body(ms_ref, h_ref, wi_ref, wh_ref, bi_ref, bh_ref, o_ref):
    ms = ms_ref[...]
    h = h_ref[...]
    cnt = ms[:, HID:HID + 1]
    inv = 1.0 / jnp.maximum(cnt, 1.0)
    m = ms * inv
    gi = jnp.dot(m, wi_ref[...], preferred_element_type=jnp.float32) + bi_ref[...]
    gh = jnp.dot(h, wh_ref[...], preferred_element_type=jnp.float32) + bh_ref[...]
    r = jax.nn.sigmoid(gi[:, :HP] + gh[:, :HP])
    z = jax.nn.sigmoid(gi[:, HP:2 * HP] + gh[:, HP:2 * HP])
    n = jnp.tanh(gi[:, 2 * HP:] + r * gh[:, 2 * HP:])
    hn = (1.0 - z) * n + z * h
    col = lax.broadcasted_iota(jnp.int32, hn.shape, 1)
    o_ref[...] = jnp.where(col == HID, 1.0, hn)


def _gru(msum, h, wi, wh, bi, bh):
    blk = 256
    return pl.pallas_call(
        _gru_body,
        grid=(NP // blk,),
        in_specs=[
            pl.BlockSpec((blk, HP), lambda i: (i, 0)),
            pl.BlockSpec((blk, HP), lambda i: (i, 0)),
            pl.BlockSpec((HP, 3 * HP), lambda i: (0, 0)),
            pl.BlockSpec((HP, 3 * HP), lambda i: (0, 0)),
            pl.BlockSpec((1, 3 * HP), lambda i: (0, 0)),
            pl.BlockSpec((1, 3 * HP), lambda i: (0, 0)),
        ],
        out_specs=pl.BlockSpec((blk, HP), lambda i: (i, 0)),
        out_shape=jax.ShapeDtypeStruct((NP, HP), jnp.float32),
    )(msum, h, wi, wh, bi, bh)


def _fc_body(g0_ref, g1_ref, w_ref, b_ref, o_ref):
    g = g0_ref[...] + g1_ref[...]
    cnt = g[:, HID:HID + 1]
    gm = g * (1.0 / jnp.maximum(cnt, 1.0))
    o_ref[...] = jnp.dot(gm, w_ref[...],
                         preferred_element_type=jnp.float32) + b_ref[...]


def _fc(g0, g1, wt, b):
    blk = 512
    return pl.pallas_call(
        _fc_body,
        grid=(VP // blk,),
        in_specs=[
            pl.BlockSpec((NUM_GRAPHS, HP), lambda j: (0, 0)),
            pl.BlockSpec((NUM_GRAPHS, HP), lambda j: (0, 0)),
            pl.BlockSpec((HP, blk), lambda j: (0, j)),
            pl.BlockSpec((1, blk), lambda j: (0, j)),
        ],
        out_specs=pl.BlockSpec((NUM_GRAPHS, blk), lambda j: (0, j)),
        out_shape=jax.ShapeDtypeStruct((NUM_GRAPHS, VP), jnp.float32),
    )(g0, g1, wt, b)


def kernel(category, sub_category, element, brand, product_id_remapped,
           price_tensor, edge_index, batch, cat_emb, sub_emb, elem_emb,
           brand_emb, item_emb, msg_W, msg_b, W_ih, W_hh, b_ih, b_hh,
           fc_W, fc_b):
    i32 = jnp.int32
    padn = NP - N

    def pad1(x, v):
        return jnp.concatenate([x.astype(i32), jnp.full((padn,), v, i32)])

    tabs = (cat_emb, sub_emb, elem_emb, brand_emb, item_emb)
    offs = []
    o = 0
    for tb in tabs:
        offs.append(o)
        o += tb.shape[0]
    stacked = jnp.concatenate(
        [jnp.pad(tb, ((0, 0), (16 * t, HP - 16 * (t + 1))))
         for t, tb in enumerate(tabs)], axis=0)

    cat_i = pad1(category, 0) + offs[0]
    sub_i = pad1(sub_category, 0) + offs[1]
    elem_i = pad1(element, 0) + offs[2]
    brand_i = pad1(brand, 0) + offs[3]
    item_i = pad1(product_id_remapped, 0) + offs[4]
    batch_i = pad1(batch, NUM_GRAPHS)
    price = jnp.pad(price_tensor, ((0, padn), (0, 0)))
    src = edge_index[0].astype(i32)
    dst = edge_index[1].astype(i32)

    zeros_h = jnp.zeros((ZROWS, HP), jnp.float32)

    wt = jnp.pad(msg_W[:, 1:].T, ((0, HP - 5 * EMB), (0, HP - HID)))
    w0 = jnp.pad(msg_W[:, 0], (0, HP - HID)).reshape(1, HP)
    b1 = jnp.pad(msg_b, (0, HP - HID)).at[HID].set(1.0).reshape(1, HP)

    def gru_w(w):
        a = jnp.pad(w.T, ((0, HP - HID), (0, 0)))
        gs = [jnp.pad(a[:, i * HID:(i + 1) * HID], ((0, 0), (0, HP - HID)))
              for i in range(3)]
        return jnp.concatenate(gs, axis=1)

    def gru_b(b):
        return jnp.concatenate(
            [jnp.pad(b[i * HID:(i + 1) * HID], (0, HP - HID))
             for i in range(3)]).reshape(1, 3 * HP)

    wi, wh = gru_w(W_ih), gru_w(W_hh)
    bi, bh = gru_b(b_ih), gru_b(b_hh)

    fct = jnp.pad(fc_W.T, ((0, HP - HID), (0, VP - fc_W.shape[0])))
    fcb = jnp.pad(fc_b, (0, VP - fc_b.shape[0])).reshape(1, VP)

    gath = _emb_gather()(cat_i, sub_i, elem_i, brand_i, item_i, stacked)
    h = _msg_linear(gath, price, wt, w0, b1)
    msum = _edge_agg()(h, src, dst, zeros_h)
    hn = _gru(msum, h, wi, wh, bi, bh)
    gparts = _pool()(hn, batch_i, zeros_h)
    scores = _fc(gparts[0, :NUM_GRAPHS], gparts[1, :NUM_GRAPHS], fct, fcb)
    return scores[:, :N]


# speedup vs baseline: 1.2041x; 1.2041x over previous
import functools

import jax
import jax.numpy as jnp
from jax import lax
from jax.experimental import pallas as pl
from jax.experimental.pallas import tpu as pltpu
from jax.experimental.pallas import tpu_sc as plsc

N = 50000
E = 800000
EMB = 16
HID = 100
NUM_GRAPHS = 512

NP = 50176
HP = 128
NPG = 640
VP = 50176

NCORES = 2
NSUB = 16
NTILES = NCORES * NSUB
ROWS_PER_TILE = NP // NTILES
GCHUNK = 112
NGCH = ROWS_PER_TILE // GCHUNK

ZROWS = 112
RANGES = 4
RSIZE = NP // RANGES
PASSES = RANGES // NCORES
ZPS = RSIZE // NSUB
ZCH = 112
NZCH = ZPS // ZCH
EPS = E // NSUB
ECH = 96
NSLOT = 2
BCH = NSLOT * ECH
NBIG = EPS // BCH
TAILE = EPS - NBIG * BCH
TAILS = []
_left = TAILE
_sl = 0
while _left > 0:
    _nv = min(_left, ECH) // 16
    TAILS.append((_sl, _nv))
    _left -= _nv * 16
    _sl += 1
IGN = -1


@functools.cache
def _mesh():
    return plsc.VectorSubcoreMesh(core_axis_name="c", subcore_axis_name="s",
                                  num_cores=NCORES, num_subcores=NSUB)


def _emb_body(i0, i1, i2, i3, i4, stacked, out_h, idx_v, rowbuf, sem):
    c = lax.axis_index("c")
    s = lax.axis_index("s")
    base = (s * NCORES + c) * ROWS_PER_TILE
    idxs = (i0, i1, i2, i3, i4)

    def body(j, _):
        r0 = base + j * GCHUNK
        pltpu.sync_copy(idxs[0].at[pl.ds(r0, GCHUNK)], idx_v)
        pltpu.async_copy(stacked.at[idx_v], rowbuf, sem).wait()
        for t in range(1, 5):
            pltpu.sync_copy(idxs[t].at[pl.ds(r0, GCHUNK)], idx_v)
            pltpu.async_copy(stacked.at[idx_v], rowbuf, sem, add=True).wait()
        pltpu.sync_copy(rowbuf, out_h.at[pl.ds(r0, GCHUNK)])
        return 0
    lax.fori_loop(0, NGCH, body, 0)


@functools.cache
def _emb_gather():
    return pl.kernel(
        _emb_body,
        out_type=jax.ShapeDtypeStruct((NP, HP), jnp.float32),
        mesh=_mesh(),
        scratch_types=[
            pltpu.VMEM((GCHUNK,), jnp.int32),
            pltpu.VMEM((GCHUNK, HP), jnp.float32),
            pltpu.SemaphoreType.DMA,
        ],
    )


def _agg_body(h_h, src_h, dst_h, zeros_h, msum_h,
              sbufL, dbufL, si0, si1, si2, si3, di0, di1, di2, di3, rows,
              g0, g1, acc):
    gs = (g0, g1)
    sis = ((si0, si1), (si2, si3))
    dis = ((di0, di1), (di2, di3))
    c = lax.axis_index("c")
    s = lax.axis_index("s")
    eb = s * EPS

    def load_big(j):
        e0 = eb + j * BCH
        pltpu.sync_copy(src_h.at[pl.ds(e0, BCH)], sbufL)
        pltpu.sync_copy(dst_h.at[pl.ds(e0, BCH)], dbufL)

    def mask_slot(lo, q, sl, nv):
        rs = sis[q][sl]
        rd = dis[q][sl]
        for i in range(nv):
            off = sl * ECH + i * 16
            d = dbufL[pl.ds(off, 16)]
            sv = sbufL[pl.ds(off, 16)]
            m = (d >= lo) & (d < lo + RSIZE)
            rs[pl.ds(i * 16, 16)] = jnp.where(m, sv, IGN)
            rd[pl.ds(i * 16, 16)] = jnp.where(m, d - lo, IGN)
        for i in range(nv, ECH // 16):
            rs[pl.ds(i * 16, 16)] = jnp.full((16,), IGN, jnp.int32)
            rd[pl.ds(i * 16, 16)] = jnp.full((16,), IGN, jnp.int32)

    def gsrc(q, sl):
        return h_h.at[plsc.Indices(sis[q][sl], ignored_value=IGN)]

    def issue_gather(q, sl):
        pltpu.async_copy(gsrc(q, sl), rows.at[sl], gs[sl])

    def wait_gather(q, sl):
        pltpu.make_async_copy(gsrc(q, sl), rows.at[sl], gs[sl]).wait()

    def do_add(q, sl):
        pltpu.sync_copy(rows.at[sl],
                        acc.at[plsc.Indices(dis[q][sl], ignored_value=IGN)],
                        add=True)

    for p in range(PASSES):
        lo = (2 * p + c) * RSIZE

        def zbody(j, _):
            pltpu.sync_copy(zeros_h.at[pl.ds(0, ZCH)],
                            acc.at[pl.ds(s * ZPS + j * ZCH, ZCH)])
            return 0
        lax.fori_loop(0, NZCH, zbody, 0)
        plsc.subcore_barrier()

        load_big(0)
        for sl in range(NSLOT):
            mask_slot(lo, 0, sl, ECH // 16)
            issue_gather(0, sl)

        def big_iter(j, q):
            load_big(j)
            for sl in range(NSLOT):
                mask_slot(lo, q, sl, ECH // 16)
            for sl in range(NSLOT):
                wait_gather(1 - q, sl)
                do_add(1 - q, sl)
                issue_gather(q, sl)

        pairs = (NBIG - 1) // 2
        def mbody(k, _):
            j = 1 + 2 * k
            big_iter(j, 1)
            big_iter(j + 1, 0)
            return 0
        lax.fori_loop(0, pairs, mbody, 0)
        if (NBIG - 1) - 2 * pairs:
            big_iter(NBIG - 1, 1)
            last_q = 1
        else:
            last_q = 0

        for sl in range(NSLOT):
            wait_gather(last_q, sl)
            do_add(last_q, sl)

        e0 = eb + NBIG * BCH
        pltpu.sync_copy(src_h.at[pl.ds(e0, TAILE)], sbufL.at[pl.ds(0, TAILE)])
        pltpu.sync_copy(dst_h.at[pl.ds(e0, TAILE)], dbufL.at[pl.ds(0, TAILE)])
        for sl, nv in TAILS:
            mask_slot(lo, 0, sl, nv)
            issue_gather(0, sl)
        for sl, nv in TAILS:
            wait_gather(0, sl)
            do_add(0, sl)
        plsc.subcore_barrier()

        pltpu.sync_copy(acc.at[pl.ds(s * ZPS, ZPS)],
                        msum_h.at[pl.ds(lo + s * ZPS, ZPS)])
        plsc.subcore_barrier()


@functools.cache
def _edge_agg():
    return pl.kernel(
        _agg_body,
        out_type=jax.ShapeDtypeStruct((NP, HP), jnp.float32),
        mesh=_mesh(),
        scratch_types=(
            [pltpu.VMEM((BCH,), jnp.int32)] * 2
            + [pltpu.VMEM((ECH,), jnp.int32)] * 8
            + [
                pltpu.VMEM((NSLOT, ECH, HP), jnp.float32),
                pltpu.SemaphoreType.DMA,
                pltpu.SemaphoreType.DMA,
                pltpu.VMEM_SHARED((RSIZE, HP), jnp.float32),
            ]
        ),
    )


def _pool_body(hn_h, batch_h, zeros_h, gpart_h, bidx, rows_v, sem, acc):
    c = lax.axis_index("c")
    s = lax.axis_index("s")
    base = (s * NCORES + c) * ROWS_PER_TILE
    gz = NPG // NSUB
    pltpu.sync_copy(zeros_h.at[pl.ds(0, gz)], acc.at[pl.ds(s * gz, gz)])
    plsc.subcore_barrier()

    def body(j, _):
        r0 = base + j * GCHUNK
        pltpu.sync_copy(batch_h.at[pl.ds(r0, GCHUNK)], bidx)
        pltpu.sync_copy(hn_h.at[pl.ds(r0, GCHUNK)], rows_v)
        pltpu.sync_copy(rows_v, acc.at[bidx], add=True)
        return 0
    lax.fori_loop(0, NGCH, body, 0)
    plsc.subcore_barrier()
    pltpu.sync_copy(acc.at[pl.ds(s * gz, gz)],
                    gpart_h.at[c, pl.ds(s * gz, gz)])


@functools.cache
def _pool():
    return pl.kernel(
        _pool_body,
        out_type=jax.ShapeDtypeStruct((NCORES, NPG, HP), jnp.float32),
        mesh=_mesh(),
        scratch_types=[
            pltpu.VMEM((GCHUNK,), jnp.int32),
            pltpu.VMEM((GCHUNK, HP), jnp.float32),
            pltpu.SemaphoreType.DMA,
            pltpu.VMEM_SHARED((NPG, HP), jnp.float32),
        ],
    )


def _mlin_body(g_ref, p_ref, wt_ref, w0_ref, b_ref, o_ref):
    acc = jnp.dot(g_ref[...], wt_ref[...], preferred_element_type=jnp.float32)
    o_ref[...] = acc + p_ref[...] * w0_ref[...] + b_ref[...]


def _msg_linear(gath, price, wt, w0, b):
    blk = 256
    return pl.pallas_call(
        _mlin_body,
        grid=(NP // blk,),
        in_specs=[
            pl.BlockSpec((blk, HP), lambda i: (i, 0)),
            pl.BlockSpec((blk, 1), lambda i: (i, 0)),
            pl.BlockSpec((HP, HP), lambda i: (0, 0)),
            pl.BlockSpec((1, HP), lambda i: (0, 0)),
            pl.BlockSpec((1, HP), lambda i: (0, 0)),
        ],
        out_specs=pl.BlockSpec((blk, HP), lambda i: (i, 0)),
        out_shape=jax.ShapeDtypeStruct((NP, HP), jnp.float32),
    )(gath, price, wt, w0, b)


def _gru_body(ms_ref, h_ref, wi_ref, wh_ref, bi_ref, bh_ref, o_ref):
    ms = ms_ref[...]
    h = h_ref[...]
    cnt = ms[:, HID:HID + 1]
    inv = 1.0 / jnp.maximum(cnt, 1.0)
    m = ms * inv
    gi = jnp.dot(m, wi_ref[...], preferred_element_type=jnp.float32) + bi_ref[...]
    gh = jnp.dot(h, wh_ref[...], preferred_element_type=jnp.float32) + bh_ref[...]
    r = jax.nn.sigmoid(gi[:, :HP] + gh[:, :HP])
    z = jax.nn.sigmoid(gi[:, HP:2 * HP] + gh[:, HP:2 * HP])
    n = jnp.tanh(gi[:, 2 * HP:] + r * gh[:, 2 * HP:])
    hn = (1.0 - z) * n + z * h
    col = lax.broadcasted_iota(jnp.int32, hn.shape, 1)
    o_ref[...] = jnp.where(col == HID, 1.0, hn)


def _gru(msum, h, wi, wh, bi, bh):
    blk = 256
    return pl.pallas_call(
        _gru_body,
        grid=(NP // blk,),
        in_specs=[
            pl.BlockSpec((blk, HP), lambda i: (i, 0)),
            pl.BlockSpec((blk, HP), lambda i: (i, 0)),
            pl.BlockSpec((HP, 3 * HP), lambda i: (0, 0)),
            pl.BlockSpec((HP, 3 * HP), lambda i: (0, 0)),
            pl.BlockSpec((1, 3 * HP), lambda i: (0, 0)),
            pl.BlockSpec((1, 3 * HP), lambda i: (0, 0)),
        ],
        out_specs=pl.BlockSpec((blk, HP), lambda i: (i, 0)),
        out_shape=jax.ShapeDtypeStruct((NP, HP), jnp.float32),
    )(msum, h, wi, wh, bi, bh)


def _fc_body(g0_ref, g1_ref, w_ref, b_ref, o_ref):
    g = g0_ref[...] + g1_ref[...]
    cnt = g[:, HID:HID + 1]
    gm = g * (1.0 / jnp.maximum(cnt, 1.0))
    o_ref[...] = jnp.dot(gm, w_ref[...],
                         preferred_element_type=jnp.float32) + b_ref[...]


def _fc(g0, g1, wt, b):
    blk = 512
    return pl.pallas_call(
        _fc_body,
        grid=(VP // blk,),
        in_specs=[
            pl.BlockSpec((NUM_GRAPHS, HP), lambda j: (0, 0)),
            pl.BlockSpec((NUM_GRAPHS, HP), lambda j: (0, 0)),
            pl.BlockSpec((HP, blk), lambda j: (0, j)),
            pl.BlockSpec((1, blk), lambda j: (0, j)),
        ],
        out_specs=pl.BlockSpec((NUM_GRAPHS, blk), lambda j: (0, j)),
        out_shape=jax.ShapeDtypeStruct((NUM_GRAPHS, VP), jnp.float32),
    )(g0, g1, wt, b)


def kernel(category, sub_category, element, brand, product_id_remapped,
           price_tensor, edge_index, batch, cat_emb, sub_emb, elem_emb,
           brand_emb, item_emb, msg_W, msg_b, W_ih, W_hh, b_ih, b_hh,
           fc_W, fc_b):
    i32 = jnp.int32
    padn = NP - N

    def pad1(x, v):
        return jnp.concatenate([x.astype(i32), jnp.full((padn,), v, i32)])

    tabs = (cat_emb, sub_emb, elem_emb, brand_emb, item_emb)
    offs = []
    o = 0
    for tb in tabs:
        offs.append(o)
        o += tb.shape[0]
    stacked = jnp.concatenate(
        [jnp.pad(tb, ((0, 0), (16 * t, HP - 16 * (t + 1))))
         for t, tb in enumerate(tabs)], axis=0)

    cat_i = pad1(category, 0) + offs[0]
    sub_i = pad1(sub_category, 0) + offs[1]
    elem_i = pad1(element, 0) + offs[2]
    brand_i = pad1(brand, 0) + offs[3]
    item_i = pad1(product_id_remapped, 0) + offs[4]
    batch_i = pad1(batch, NUM_GRAPHS)
    price = jnp.pad(price_tensor, ((0, padn), (0, 0)))
    src = edge_index[0].astype(i32)
    dst = edge_index[1].astype(i32)

    zeros_h = jnp.zeros((ZROWS, HP), jnp.float32)

    wt = jnp.pad(msg_W[:, 1:].T, ((0, HP - 5 * EMB), (0, HP - HID)))
    w0 = jnp.pad(msg_W[:, 0], (0, HP - HID)).reshape(1, HP)
    b1 = jnp.pad(msg_b, (0, HP - HID)).at[HID].set(1.0).reshape(1, HP)

    def gru_w(w):
        a = jnp.pad(w.T, ((0, HP - HID), (0, 0)))
        gs = [jnp.pad(a[:, i * HID:(i + 1) * HID], ((0, 0), (0, HP - HID)))
              for i in range(3)]
        return jnp.concatenate(gs, axis=1)

    def gru_b(b):
        return jnp.concatenate(
            [jnp.pad(b[i * HID:(i + 1) * HID], (0, HP - HID))
             for i in range(3)]).reshape(1, 3 * HP)

    wi, wh = gru_w(W_ih), gru_w(W_hh)
    bi, bh = gru_b(b_ih), gru_b(b_hh)

    fct = jnp.pad(fc_W.T, ((0, HP - HID), (0, VP - fc_W.shape[0])))
    fcb = jnp.pad(fc_b, (0, VP - fc_b.shape[0])).reshape(1, VP)

    gath = _emb_gather()(cat_i, sub_i, elem_i, brand_i, item_i, stacked)
    h = _msg_linear(gath, price, wt, w0, b1)
    msum = _edge_agg()(h, src, dst, zeros_h)
    hn = _gru(msum, h, wi, wh, bi, bh)
    gparts = _pool()(hn, batch_i, zeros_h)
    scores = _fc(gparts[0, :NUM_GRAPHS], gparts[1, :NUM_GRAPHS], fct, fcb)
    return scores[:, :N]


# speedup vs baseline: 1.3736x; 1.1408x over previous
import functools

import jax
import jax.numpy as jnp
from jax import lax
from jax.experimental import pallas as pl
from jax.experimental.pallas import tpu as pltpu
from jax.experimental.pallas import tpu_sc as plsc

N = 50000
E = 800000
EMB = 16
HID = 100
NUM_GRAPHS = 512

NP = 50176
HP = 128
NPG = 640
VP = 50176

NCORES = 2
NSUB = 16
NTILES = NCORES * NSUB
ROWS_PER_TILE = NP // NTILES
GCHUNK = 112
NGCH = ROWS_PER_TILE // GCHUNK

ZROWS = 112
RANGES = 4
RSIZE = NP // RANGES
PASSES = RANGES // NCORES
ZPS = RSIZE // NSUB
ZCH = 112
NZCH = ZPS // ZCH
EPS = E // NSUB
ECH = 96
NSLOT = 2
BCH = NSLOT * ECH
NBIG = EPS // BCH
TAILE = EPS - NBIG * BCH
TAILS = []
_left = TAILE
_sl = 0
while _left > 0:
    _nv = min(_left, ECH) // 16
    TAILS.append((_sl, _nv))
    _left -= _nv * 16
    _sl += 1
IGN = -1


@functools.cache
def _mesh():
    return plsc.VectorSubcoreMesh(core_axis_name="c", subcore_axis_name="s",
                                  num_cores=NCORES, num_subcores=NSUB)


def _emb_body(i0, i1, i2, i3, i4, stacked, out_h, idx_v, rowbuf, sem):
    c = lax.axis_index("c")
    s = lax.axis_index("s")
    base = (s * NCORES + c) * ROWS_PER_TILE
    idxs = (i0, i1, i2, i3, i4)

    def body(j, _):
        r0 = base + j * GCHUNK
        pltpu.sync_copy(idxs[0].at[pl.ds(r0, GCHUNK)], idx_v)
        pltpu.async_copy(stacked.at[idx_v], rowbuf, sem).wait()
        for t in range(1, 5):
            pltpu.sync_copy(idxs[t].at[pl.ds(r0, GCHUNK)], idx_v)
            pltpu.async_copy(stacked.at[idx_v], rowbuf, sem, add=True).wait()
        pltpu.sync_copy(rowbuf, out_h.at[pl.ds(r0, GCHUNK)])
        return 0
    lax.fori_loop(0, NGCH, body, 0)


@functools.cache
def _emb_gather():
    return pl.kernel(
        _emb_body,
        out_type=jax.ShapeDtypeStruct((NP, HP), jnp.float32),
        mesh=_mesh(),
        scratch_types=[
            pltpu.VMEM((GCHUNK,), jnp.int32),
            pltpu.VMEM((GCHUNK, HP), jnp.float32),
            pltpu.SemaphoreType.DMA,
        ],
    )


def _agg_body(h_h, src_h, dst_h, zeros_h, msum_h,
              sb0, sb1, db0, db1, si0, si1, si2, si3, di0, di1, di2, di3,
              rows, g0, g1, i0s, i0d, i1s, i1d, acc):
    gs = (g0, g1)
    iss = (i0s, i1s)
    isd = (i0d, i1d)
    sbs = (sb0, sb1)
    dbs = (db0, db1)
    sis = ((si0, si1), (si2, si3))
    dis = ((di0, di1), (di2, di3))
    c = lax.axis_index("c")
    s = lax.axis_index("s")
    eb = s * EPS

    def idx_refs(j, q):
        e0 = jnp.minimum(eb + j * BCH, eb + EPS - BCH)
        return ((src_h.at[pl.ds(e0, BCH)], sbs[q], iss[q]),
                (dst_h.at[pl.ds(e0, BCH)], dbs[q], isd[q]))

    def issue_idx(j, q):
        for tr in idx_refs(j, q):
            pltpu.async_copy(*tr)

    def wait_idx(j, q):
        for tr in idx_refs(j, q):
            pltpu.make_async_copy(*tr).wait()

    def mask_slot(lo, q, sl, nv):
        rs = sis[q][sl]
        rd = dis[q][sl]
        sb = sbs[q]
        db = dbs[q]
        for i in range(nv):
            off = sl * ECH + i * 16
            d = db[pl.ds(off, 16)]
            sv = sb[pl.ds(off, 16)]
            m = (d >= lo) & (d < lo + RSIZE)
            rs[pl.ds(i * 16, 16)] = jnp.where(m, sv, IGN)
            rd[pl.ds(i * 16, 16)] = jnp.where(m, d - lo, IGN)
        for i in range(nv, ECH // 16):
            rs[pl.ds(i * 16, 16)] = jnp.full((16,), IGN, jnp.int32)
            rd[pl.ds(i * 16, 16)] = jnp.full((16,), IGN, jnp.int32)

    def gsrc(q, sl):
        return h_h.at[plsc.Indices(sis[q][sl], ignored_value=IGN)]

    def issue_gather(q, sl):
        pltpu.async_copy(gsrc(q, sl), rows.at[sl], gs[sl])

    def wait_gather(q, sl):
        pltpu.make_async_copy(gsrc(q, sl), rows.at[sl], gs[sl]).wait()

    def do_add(q, sl):
        pltpu.sync_copy(rows.at[sl],
                        acc.at[plsc.Indices(dis[q][sl], ignored_value=IGN)],
                        add=True)

    for p in range(PASSES):
        lo = (2 * p + c) * RSIZE

        def zbody(j, _):
            pltpu.sync_copy(zeros_h.at[pl.ds(0, ZCH)],
                            acc.at[pl.ds(s * ZPS + j * ZCH, ZCH)])
            return 0
        lax.fori_loop(0, NZCH, zbody, 0)
        plsc.subcore_barrier()

        issue_idx(0, 0)
        wait_idx(0, 0)
        issue_idx(1, 1)
        for sl in range(NSLOT):
            mask_slot(lo, 0, sl, ECH // 16)
            issue_gather(0, sl)

        def big_iter(j, q):
            wait_idx(j, q)
            issue_idx(j + 1, 1 - q)
            for sl in range(NSLOT):
                mask_slot(lo, q, sl, ECH // 16)
            for sl in range(NSLOT):
                wait_gather(1 - q, sl)
                do_add(1 - q, sl)
                issue_gather(q, sl)

        pairs = (NBIG - 1) // 2
        def mbody(k, _):
            j = 1 + 2 * k
            big_iter(j, 1)
            big_iter(j + 1, 0)
            return 0
        lax.fori_loop(0, pairs, mbody, 0)
        if (NBIG - 1) - 2 * pairs:
            big_iter(NBIG - 1, 1)
            last_q = 1
        else:
            last_q = 0

        for sl in range(NSLOT):
            wait_gather(last_q, sl)
            do_add(last_q, sl)
        wait_idx(NBIG, 1 - last_q)

        e0 = eb + NBIG * BCH
        pltpu.sync_copy(src_h.at[pl.ds(e0, TAILE)], sb0.at[pl.ds(0, TAILE)])
        pltpu.sync_copy(dst_h.at[pl.ds(e0, TAILE)], db0.at[pl.ds(0, TAILE)])
        for sl, nv in TAILS:
            mask_slot(lo, 0, sl, nv)
            issue_gather(0, sl)
        for sl, nv in TAILS:
            wait_gather(0, sl)
            do_add(0, sl)
        plsc.subcore_barrier()

        pltpu.sync_copy(acc.at[pl.ds(s * ZPS, ZPS)],
                        msum_h.at[pl.ds(lo + s * ZPS, ZPS)])
        plsc.subcore_barrier()


@functools.cache
def _edge_agg():
    return pl.kernel(
        _agg_body,
        out_type=jax.ShapeDtypeStruct((NP, HP), jnp.float32),
        mesh=_mesh(),
        scratch_types=(
            [pltpu.VMEM((BCH,), jnp.int32)] * 4
            + [pltpu.VMEM((ECH,), jnp.int32)] * 8
            + [pltpu.VMEM((NSLOT, ECH, HP), jnp.float32)]
            + [pltpu.SemaphoreType.DMA] * 6
            + [pltpu.VMEM_SHARED((RSIZE, HP), jnp.float32)]
        ),
    )


def _pool_body(hn_h, batch_h, zeros_h, gpart_h, bidx, rows_v, sem, acc):
    c = lax.axis_index("c")
    s = lax.axis_index("s")
    base = (s * NCORES + c) * ROWS_PER_TILE
    gz = NPG // NSUB
    pltpu.sync_copy(zeros_h.at[pl.ds(0, gz)], acc.at[pl.ds(s * gz, gz)])
    plsc.subcore_barrier()

    def body(j, _):
        r0 = base + j * GCHUNK
        pltpu.sync_copy(batch_h.at[pl.ds(r0, GCHUNK)], bidx)
        pltpu.sync_copy(hn_h.at[pl.ds(r0, GCHUNK)], rows_v)
        pltpu.sync_copy(rows_v, acc.at[bidx], add=True)
        return 0
    lax.fori_loop(0, NGCH, body, 0)
    plsc.subcore_barrier()
    pltpu.sync_copy(acc.at[pl.ds(s * gz, gz)],
                    gpart_h.at[c, pl.ds(s * gz, gz)])


@functools.cache
def _pool():
    return pl.kernel(
        _pool_body,
        out_type=jax.ShapeDtypeStruct((NCORES, NPG, HP), jnp.float32),
        mesh=_mesh(),
        scratch_types=[
            pltpu.VMEM((GCHUNK,), jnp.int32),
            pltpu.VMEM((GCHUNK, HP), jnp.float32),
            pltpu.SemaphoreType.DMA,
            pltpu.VMEM_SHARED((NPG, HP), jnp.float32),
        ],
    )


def _mlin_body(g_ref, p_ref, wt_ref, w0_ref, b_ref, o_ref):
    acc = jnp.dot(g_ref[...], wt_ref[...], preferred_element_type=jnp.float32)
    o_ref[...] = acc + p_ref[...] * w0_ref[...] + b_ref[...]


def _msg_linear(gath, price, wt, w0, b):
    blk = 256
    return pl.pallas_call(
        _mlin_body,
        grid=(NP // blk,),
        in_specs=[
            pl.BlockSpec((blk, HP), lambda i: (i, 0)),
            pl.BlockSpec((blk, 1), lambda i: (i, 0)),
            pl.BlockSpec((HP, HP), lambda i: (0, 0)),
            pl.BlockSpec((1, HP), lambda i: (0, 0)),
            pl.BlockSpec((1, HP), lambda i: (0, 0)),
        ],
        out_specs=pl.BlockSpec((blk, HP), lambda i: (i, 0)),
        out_shape=jax.ShapeDtypeStruct((NP, HP), jnp.float32),
    )(gath, price, wt, w0, b)


def _gru_body(ms_ref, h_ref, wi_ref, wh_ref, bi_ref, bh_ref, o_ref):
    ms = ms_ref[...]
    h = h_ref[...]
    cnt = ms[:, HID:HID + 1]
    inv = 1.0 / jnp.maximum(cnt, 1.0)
    m = ms * inv
    gi = jnp.dot(m, wi_ref[...], preferred_element_type=jnp.float32) + bi_ref[...]
    gh = jnp.dot(h, wh_ref[...], preferred_element_type=jnp.float32) + bh_ref[...]
    r = jax.nn.sigmoid(gi[:, :HP] + gh[:, :HP])
    z = jax.nn.sigmoid(gi[:, HP:2 * HP] + gh[:, HP:2 * HP])
    n = jnp.tanh(gi[:, 2 * HP:] + r * gh[:, 2 * HP:])
    hn = (1.0 - z) * n + z * h
    col = lax.broadcasted_iota(jnp.int32, hn.shape, 1)
    o_ref[...] = jnp.where(col == HID, 1.0, hn)


def _gru(msum, h, wi, wh, bi, bh):
    blk = 256
    return pl.pallas_call(
        _gru_body,
        grid=(NP // blk,),
        in_specs=[
            pl.BlockSpec((blk, HP), lambda i: (i, 0)),
            pl.BlockSpec((blk, HP), lambda i: (i, 0)),
            pl.BlockSpec((HP, 3 * HP), lambda i: (0, 0)),
            pl.BlockSpec((HP, 3 * HP), lambda i: (0, 0)),
            pl.BlockSpec((1, 3 * HP), lambda i: (0, 0)),
            pl.BlockSpec((1, 3 * HP), lambda i: (0, 0)),
        ],
        out_specs=pl.BlockSpec((blk, HP), lambda i: (i, 0)),
        out_shape=jax.ShapeDtypeStruct((NP, HP), jnp.float32),
    )(msum, h, wi, wh, bi, bh)


def _fc_body(g0_ref, g1_ref, w_ref, b_ref, o_ref):
    g = g0_ref[...] + g1_ref[...]
    cnt = g[:, HID:HID + 1]
    gm = g * (1.0 / jnp.maximum(cnt, 1.0))
    o_ref[...] = jnp.dot(gm, w_ref[...],
                         preferred_element_type=jnp.float32) + b_ref[...]


def _fc(g0, g1, wt, b):
    blk = 512
    return pl.pallas_call(
        _fc_body,
        grid=(VP // blk,),
        in_specs=[
            pl.BlockSpec((NUM_GRAPHS, HP), lambda j: (0, 0)),
            pl.BlockSpec((NUM_GRAPHS, HP), lambda j: (0, 0)),
            pl.BlockSpec((HP, blk), lambda j: (0, j)),
            pl.BlockSpec((1, blk), lambda j: (0, j)),
        ],
        out_specs=pl.BlockSpec((NUM_GRAPHS, blk), lambda j: (0, j)),
        out_shape=jax.ShapeDtypeStruct((NUM_GRAPHS, VP), jnp.float32),
    )(g0, g1, wt, b)


def kernel(category, sub_category, element, brand, product_id_remapped,
           price_tensor, edge_index, batch, cat_emb, sub_emb, elem_emb,
           brand_emb, item_emb, msg_W, msg_b, W_ih, W_hh, b_ih, b_hh,
           fc_W, fc_b):
    i32 = jnp.int32
    padn = NP - N

    def pad1(x, v):
        return jnp.concatenate([x.astype(i32), jnp.full((padn,), v, i32)])

    tabs = (cat_emb, sub_emb, elem_emb, brand_emb, item_emb)
    offs = []
    o = 0
    for tb in tabs:
        offs.append(o)
        o += tb.shape[0]
    stacked = jnp.concatenate(
        [jnp.pad(tb, ((0, 0), (16 * t, HP - 16 * (t + 1))))
         for t, tb in enumerate(tabs)], axis=0)

    cat_i = pad1(category, 0) + offs[0]
    sub_i = pad1(sub_category, 0) + offs[1]
    elem_i = pad1(element, 0) + offs[2]
    brand_i = pad1(brand, 0) + offs[3]
    item_i = pad1(product_id_remapped, 0) + offs[4]
    batch_i = pad1(batch, NUM_GRAPHS)
    price = jnp.pad(price_tensor, ((0, padn), (0, 0)))
    src = edge_index[0].astype(i32)
    dst = edge_index[1].astype(i32)

    zeros_h = jnp.zeros((ZROWS, HP), jnp.float32)

    wt = jnp.pad(msg_W[:, 1:].T, ((0, HP - 5 * EMB), (0, HP - HID)))
    w0 = jnp.pad(msg_W[:, 0], (0, HP - HID)).reshape(1, HP)
    b1 = jnp.pad(msg_b, (0, HP - HID)).at[HID].set(1.0).reshape(1, HP)

    def gru_w(w):
        a = jnp.pad(w.T, ((0, HP - HID), (0, 0)))
        gs = [jnp.pad(a[:, i * HID:(i + 1) * HID], ((0, 0), (0, HP - HID)))
              for i in range(3)]
        return jnp.concatenate(gs, axis=1)

    def gru_b(b):
        return jnp.concatenate(
            [jnp.pad(b[i * HID:(i + 1) * HID], (0, HP - HID))
             for i in range(3)]).reshape(1, 3 * HP)

    wi, wh = gru_w(W_ih), gru_w(W_hh)
    bi, bh = gru_b(b_ih), gru_b(b_hh)

    fct = jnp.pad(fc_W.T, ((0, HP - HID), (0, VP - fc_W.shape[0])))
    fcb = jnp.pad(fc_b, (0, VP - fc_b.shape[0])).reshape(1, VP)

    gath = _emb_gather()(cat_i, sub_i, elem_i, brand_i, item_i, stacked)
    h = _msg_linear(gath, price, wt, w0, b1)
    msum = _edge_agg()(h, src, dst, zeros_h)
    hn = _gru(msum, h, wi, wh, bi, bh)
    gparts = _pool()(hn, batch_i, zeros_h)
    scores = _fc(gparts[0, :NUM_GRAPHS], gparts[1, :NUM_GRAPHS], fct, fcb)
    return scores[:, :N]


# speedup vs baseline: 1.4138x; 1.0293x over previous
import functools

import jax
import jax.numpy as jnp
from jax import lax
from jax.experimental import pallas as pl
from jax.experimental.pallas import tpu as pltpu
from jax.experimental.pallas import tpu_sc as plsc

N = 50000
E = 800000
EMB = 16
HID = 100
NUM_GRAPHS = 512

NP = 50176
HP = 128
NPG = 640
VP = 50176

NCORES = 2
NSUB = 16
NTILES = NCORES * NSUB
ROWS_PER_TILE = NP // NTILES
GCHUNK = 112
NGCH = ROWS_PER_TILE // GCHUNK

ZROWS = 112
RANGES = 4
RSIZE = NP // RANGES
PASSES = RANGES // NCORES
ZPS = RSIZE // NSUB
ZCH = 112
NZCH = ZPS // ZCH
EPS = E // NSUB
ECH = 112
NSLOT = 2
BCH = NSLOT * ECH
NBIG = EPS // BCH
TAILE = EPS - NBIG * BCH
TAILS = []
_left = TAILE
_sl = 0
while _left > 0:
    _nv = min(_left, ECH) // 16
    TAILS.append((_sl, _nv))
    _left -= _nv * 16
    _sl += 1
IGN = -1


@functools.cache
def _mesh():
    return plsc.VectorSubcoreMesh(core_axis_name="c", subcore_axis_name="s",
                                  num_cores=NCORES, num_subcores=NSUB)


def _emb_body(i0, i1, i2, i3, i4, stacked, out_h, idx_v, rowbuf, sem):
    c = lax.axis_index("c")
    s = lax.axis_index("s")
    base = (s * NCORES + c) * ROWS_PER_TILE
    idxs = (i0, i1, i2, i3, i4)

    def body(j, _):
        r0 = base + j * GCHUNK
        pltpu.sync_copy(idxs[0].at[pl.ds(r0, GCHUNK)], idx_v)
        pltpu.async_copy(stacked.at[idx_v], rowbuf, sem).wait()
        for t in range(1, 5):
            pltpu.sync_copy(idxs[t].at[pl.ds(r0, GCHUNK)], idx_v)
            pltpu.async_copy(stacked.at[idx_v], rowbuf, sem, add=True).wait()
        pltpu.sync_copy(rowbuf, out_h.at[pl.ds(r0, GCHUNK)])
        return 0
    lax.fori_loop(0, NGCH, body, 0)


@functools.cache
def _emb_gather():
    return pl.kernel(
        _emb_body,
        out_type=jax.ShapeDtypeStruct((NP, HP), jnp.float32),
        mesh=_mesh(),
        scratch_types=[
            pltpu.VMEM((GCHUNK,), jnp.int32),
            pltpu.VMEM((GCHUNK, HP), jnp.float32),
            pltpu.SemaphoreType.DMA,
        ],
    )


def _agg_body(h_h, src_h, dst_h, zeros_h, msum_h,
              sb0, sb1, db0, db1, si0, si1, si2, si3, di0, di1, di2, di3,
              rows, g0, g1, i0s, i0d, i1s, i1d, acc):
    gs = (g0, g1)
    iss = (i0s, i1s)
    isd = (i0d, i1d)
    sbs = (sb0, sb1)
    dbs = (db0, db1)
    sis = ((si0, si1), (si2, si3))
    dis = ((di0, di1), (di2, di3))
    c = lax.axis_index("c")
    s = lax.axis_index("s")
    eb = s * EPS

    def idx_refs(j, q):
        e0 = jnp.minimum(eb + j * BCH, eb + EPS - BCH)
        return ((src_h.at[pl.ds(e0, BCH)], sbs[q], iss[q]),
                (dst_h.at[pl.ds(e0, BCH)], dbs[q], isd[q]))

    def issue_idx(j, q):
        for tr in idx_refs(j, q):
            pltpu.async_copy(*tr)

    def wait_idx(j, q):
        for tr in idx_refs(j, q):
            pltpu.make_async_copy(*tr).wait()

    def mask_slot(lo, q, sl, nv):
        rs = sis[q][sl]
        rd = dis[q][sl]
        sb = sbs[q]
        db = dbs[q]
        for i in range(nv):
            off = sl * ECH + i * 16
            d = db[pl.ds(off, 16)]
            sv = sb[pl.ds(off, 16)]
            m = (d >= lo) & (d < lo + RSIZE)
            rs[pl.ds(i * 16, 16)] = jnp.where(m, sv, IGN)
            rd[pl.ds(i * 16, 16)] = jnp.where(m, d - lo, IGN)
        for i in range(nv, ECH // 16):
            rs[pl.ds(i * 16, 16)] = jnp.full((16,), IGN, jnp.int32)
            rd[pl.ds(i * 16, 16)] = jnp.full((16,), IGN, jnp.int32)

    def gsrc(q, sl):
        return h_h.at[plsc.Indices(sis[q][sl], ignored_value=IGN)]

    def issue_gather(q, sl):
        pltpu.async_copy(gsrc(q, sl), rows.at[sl], gs[sl])

    def wait_gather(q, sl):
        pltpu.make_async_copy(gsrc(q, sl), rows.at[sl], gs[sl]).wait()

    def do_add(q, sl):
        pltpu.sync_copy(rows.at[sl],
                        acc.at[plsc.Indices(dis[q][sl], ignored_value=IGN)],
                        add=True)

    for p in range(PASSES):
        lo = (2 * p + c) * RSIZE

        def zbody(j, _):
            pltpu.sync_copy(zeros_h.at[pl.ds(0, ZCH)],
                            acc.at[pl.ds(s * ZPS + j * ZCH, ZCH)])
            return 0
        lax.fori_loop(0, NZCH, zbody, 0)
        plsc.subcore_barrier()

        issue_idx(0, 0)
        wait_idx(0, 0)
        issue_idx(1, 1)
        for sl in range(NSLOT):
            mask_slot(lo, 0, sl, ECH // 16)
            issue_gather(0, sl)

        def big_iter(j, q):
            wait_idx(j, q)
            issue_idx(j + 1, 1 - q)
            for sl in range(NSLOT):
                mask_slot(lo, q, sl, ECH // 16)
            for sl in range(NSLOT):
                wait_gather(1 - q, sl)
                do_add(1 - q, sl)
                issue_gather(q, sl)

        pairs = (NBIG - 1) // 2
        def mbody(k, _):
            j = 1 + 2 * k
            big_iter(j, 1)
            big_iter(j + 1, 0)
            return 0
        lax.fori_loop(0, pairs, mbody, 0)
        if (NBIG - 1) - 2 * pairs:
            big_iter(NBIG - 1, 1)
            last_q = 1
        else:
            last_q = 0

        for sl in range(NSLOT):
            wait_gather(last_q, sl)
            do_add(last_q, sl)
        wait_idx(NBIG, 1 - last_q)

        e0 = eb + NBIG * BCH
        pltpu.sync_copy(src_h.at[pl.ds(e0, TAILE)], sb0.at[pl.ds(0, TAILE)])
        pltpu.sync_copy(dst_h.at[pl.ds(e0, TAILE)], db0.at[pl.ds(0, TAILE)])
        for sl, nv in TAILS:
            mask_slot(lo, 0, sl, nv)
            issue_gather(0, sl)
        for sl, nv in TAILS:
            wait_gather(0, sl)
            do_add(0, sl)
        plsc.subcore_barrier()

        pltpu.sync_copy(acc.at[pl.ds(s * ZPS, ZPS)],
                        msum_h.at[pl.ds(lo + s * ZPS, ZPS)])
        plsc.subcore_barrier()


@functools.cache
def _edge_agg():
    return pl.kernel(
        _agg_body,
        out_type=jax.ShapeDtypeStruct((NP, HP), jnp.float32),
        mesh=_mesh(),
        scratch_types=(
            [pltpu.VMEM((BCH,), jnp.int32)] * 4
            + [pltpu.VMEM((ECH,), jnp.int32)] * 8
            + [pltpu.VMEM((NSLOT, ECH, HP), jnp.float32)]
            + [pltpu.SemaphoreType.DMA] * 6
            + [pltpu.VMEM_SHARED((RSIZE, HP), jnp.float32)]
        ),
    )


def _pool_body(hn_h, batch_h, zeros_h, gpart_h, bidx, rows_v, sem, acc):
    c = lax.axis_index("c")
    s = lax.axis_index("s")
    base = (s * NCORES + c) * ROWS_PER_TILE
    gz = NPG // NSUB
    pltpu.sync_copy(zeros_h.at[pl.ds(0, gz)], acc.at[pl.ds(s * gz, gz)])
    plsc.subcore_barrier()

    def body(j, _):
        r0 = base + j * GCHUNK
        pltpu.sync_copy(batch_h.at[pl.ds(r0, GCHUNK)], bidx)
        pltpu.sync_copy(hn_h.at[pl.ds(r0, GCHUNK)], rows_v)
        pltpu.sync_copy(rows_v, acc.at[bidx], add=True)
        return 0
    lax.fori_loop(0, NGCH, body, 0)
    plsc.subcore_barrier()
    pltpu.sync_copy(acc.at[pl.ds(s * gz, gz)],
                    gpart_h.at[c, pl.ds(s * gz, gz)])


@functools.cache
def _pool():
    return pl.kernel(
        _pool_body,
        out_type=jax.ShapeDtypeStruct((NCORES, NPG, HP), jnp.float32),
        mesh=_mesh(),
        scratch_types=[
            pltpu.VMEM((GCHUNK,), jnp.int32),
            pltpu.VMEM((GCHUNK, HP), jnp.float32),
            pltpu.SemaphoreType.DMA,
            pltpu.VMEM_SHARED((NPG, HP), jnp.float32),
        ],
    )


def _mlin_body(g_ref, p_ref, wt_ref, w0_ref, b_ref, o_ref):
    acc = jnp.dot(g_ref[...], wt_ref[...], preferred_element_type=jnp.float32)
    o_ref[...] = acc + p_ref[...] * w0_ref[...] + b_ref[...]


def _msg_linear(gath, price, wt, w0, b):
    blk = 256
    return pl.pallas_call(
        _mlin_body,
        grid=(NP // blk,),
        in_specs=[
            pl.BlockSpec((blk, HP), lambda i: (i, 0)),
            pl.BlockSpec((blk, 1), lambda i: (i, 0)),
            pl.BlockSpec((HP, HP), lambda i: (0, 0)),
            pl.BlockSpec((1, HP), lambda i: (0, 0)),
            pl.BlockSpec((1, HP), lambda i: (0, 0)),
        ],
        out_specs=pl.BlockSpec((blk, HP), lambda i: (i, 0)),
        out_shape=jax.ShapeDtypeStruct((NP, HP), jnp.float32),
    )(gath, price, wt, w0, b)


def _gru_body(ms_ref, h_ref, wi_ref, wh_ref, bi_ref, bh_ref, o_ref):
    ms = ms_ref[...]
    h = h_ref[...]
    cnt = ms[:, HID:HID + 1]
    inv = 1.0 / jnp.maximum(cnt, 1.0)
    m = ms * inv
    gi = jnp.dot(m, wi_ref[...], preferred_element_type=jnp.float32) + bi_ref[...]
    gh = jnp.dot(h, wh_ref[...], preferred_element_type=jnp.float32) + bh_ref[...]
    r = jax.nn.sigmoid(gi[:, :HP] + gh[:, :HP])
    z = jax.nn.sigmoid(gi[:, HP:2 * HP] + gh[:, HP:2 * HP])
    n = jnp.tanh(gi[:, 2 * HP:] + r * gh[:, 2 * HP:])
    hn = (1.0 - z) * n + z * h
    col = lax.broadcasted_iota(jnp.int32, hn.shape, 1)
    o_ref[...] = jnp.where(col == HID, 1.0, hn)


def _gru(msum, h, wi, wh, bi, bh):
    blk = 256
    return pl.pallas_call(
        _gru_body,
        grid=(NP // blk,),
        in_specs=[
            pl.BlockSpec((blk, HP), lambda i: (i, 0)),
            pl.BlockSpec((blk, HP), lambda i: (i, 0)),
            pl.BlockSpec((HP, 3 * HP), lambda i: (0, 0)),
            pl.BlockSpec((HP, 3 * HP), lambda i: (0, 0)),
            pl.BlockSpec((1, 3 * HP), lambda i: (0, 0)),
            pl.BlockSpec((1, 3 * HP), lambda i: (0, 0)),
        ],
        out_specs=pl.BlockSpec((blk, HP), lambda i: (i, 0)),
        out_shape=jax.ShapeDtypeStruct((NP, HP), jnp.float32),
    )(msum, h, wi, wh, bi, bh)


def _fc_body(g0_ref, g1_ref, w_ref, b_ref, o_ref):
    g = g0_ref[...] + g1_ref[...]
    cnt = g[:, HID:HID + 1]
    gm = g * (1.0 / jnp.maximum(cnt, 1.0))
    o_ref[...] = jnp.dot(gm, w_ref[...],
                         preferred_element_type=jnp.float32) + b_ref[...]


def _fc(g0, g1, wt, b):
    blk = 512
    return pl.pallas_call(
        _fc_body,
        grid=(VP // blk,),
        in_specs=[
            pl.BlockSpec((NUM_GRAPHS, HP), lambda j: (0, 0)),
            pl.BlockSpec((NUM_GRAPHS, HP), lambda j: (0, 0)),
            pl.BlockSpec((HP, blk), lambda j: (0, j)),
            pl.BlockSpec((1, blk), lambda j: (0, j)),
        ],
        out_specs=pl.BlockSpec((NUM_GRAPHS, blk), lambda j: (0, j)),
        out_shape=jax.ShapeDtypeStruct((NUM_GRAPHS, VP), jnp.float32),
    )(g0, g1, wt, b)


def kernel(category, sub_category, element, brand, product_id_remapped,
           price_tensor, edge_index, batch, cat_emb, sub_emb, elem_emb,
           brand_emb, item_emb, msg_W, msg_b, W_ih, W_hh, b_ih, b_hh,
           fc_W, fc_b):
    i32 = jnp.int32
    padn = NP - N

    def pad1(x, v):
        return jnp.concatenate([x.astype(i32), jnp.full((padn,), v, i32)])

    tabs = (cat_emb, sub_emb, elem_emb, brand_emb, item_emb)
    offs = []
    o = 0
    for tb in tabs:
        offs.append(o)
        o += tb.shape[0]
    stacked = jnp.concatenate(
        [jnp.pad(tb, ((0, 0), (16 * t, HP - 16 * (t + 1))))
         for t, tb in enumerate(tabs)], axis=0)

    cat_i = pad1(category, 0) + offs[0]
    sub_i = pad1(sub_category, 0) + offs[1]
    elem_i = pad1(element, 0) + offs[2]
    brand_i = pad1(brand, 0) + offs[3]
    item_i = pad1(product_id_remapped, 0) + offs[4]
    batch_i = pad1(batch, NUM_GRAPHS)
    price = jnp.pad(price_tensor, ((0, padn), (0, 0)))
    src = edge_index[0].astype(i32)
    dst = edge_index[1].astype(i32)

    zeros_h = jnp.zeros((ZROWS, HP), jnp.float32)

    wt = jnp.pad(msg_W[:, 1:].T, ((0, HP - 5 * EMB), (0, HP - HID)))
    w0 = jnp.pad(msg_W[:, 0], (0, HP - HID)).reshape(1, HP)
    b1 = jnp.pad(msg_b, (0, HP - HID)).at[HID].set(1.0).reshape(1, HP)

    def gru_w(w):
        a = jnp.pad(w.T, ((0, HP - HID), (0, 0)))
        gs = [jnp.pad(a[:, i * HID:(i + 1) * HID], ((0, 0), (0, HP - HID)))
              for i in range(3)]
        return jnp.concatenate(gs, axis=1)

    def gru_b(b):
        return jnp.concatenate(
            [jnp.pad(b[i * HID:(i + 1) * HID], (0, HP - HID))
             for i in range(3)]).reshape(1, 3 * HP)

    wi, wh = gru_w(W_ih), gru_w(W_hh)
    bi, bh = gru_b(b_ih), gru_b(b_hh)

    fct = jnp.pad(fc_W.T, ((0, HP - HID), (0, VP - fc_W.shape[0])))
    fcb = jnp.pad(fc_b, (0, VP - fc_b.shape[0])).reshape(1, VP)

    gath = _emb_gather()(cat_i, sub_i, elem_i, brand_i, item_i, stacked)
    h = _msg_linear(gath, price, wt, w0, b1)
    msum = _edge_agg()(h, src, dst, zeros_h)
    hn = _gru(msum, h, wi, wh, bi, bh)
    gparts = _pool()(hn, batch_i, zeros_h)
    scores = _fc(gparts[0, :NUM_GRAPHS], gparts[1, :NUM_GRAPHS], fct, fcb)
    return scores[:, :N]
